# Initial kernel scaffold; baseline (speedup 1.0000x reference)
import jax
import jax.numpy as jnp
from jax.experimental import pallas as pl


def kernel(x, edge_index, edge_type, W1, b1, W2, b2, Wc1, bc1, Wc2, bc2):
    raise NotImplementedError("write your pallas kernel here")


# speedup vs baseline: 1.5582x; 1.5582x over previous
import functools

import jax
import jax.numpy as jnp
from jax import lax
from jax.experimental import pallas as pl
from jax.experimental.pallas import tpu as pltpu
from jax.experimental.pallas import tpu_sc as plsc

N = 10000
E = 160000
EP = 163840
NP = 10240
EMB = 256
FC = 512
HALF = 5000
CE = 128

_mesh = plsc.VectorSubcoreMesh(core_axis_name="c", subcore_axis_name="s")


@functools.partial(
    pl.kernel, mesh=_mesh,
    compiler_params=pltpu.CompilerParams(needs_layout_passes=False),
    out_type=jax.ShapeDtypeStruct((2 * 4 * NP,), jnp.float32),
    scratch_types=[
        pltpu.VMEM((5120,), jnp.int32),
        pltpu.VMEM((5120,), jnp.int32),
        pltpu.VMEM((128,), jnp.int32),
        pltpu.VMEM((128,), jnp.float32),
        pltpu.VMEM_SHARED((4 * NP,), jnp.float32),
        pltpu.SemaphoreType.DMA,
    ])
def _deg_kernel(dst_hbm, typ_hbm, z_hbm, cnt_hbm, dbuf, tbuf, ibuf, ones, acc, sem):
    c = lax.axis_index("c")
    s = lax.axis_index("s")
    tile = s * 2 + c
    base = tile * 5120
    pltpu.sync_copy(z_hbm.at[pl.ds(s * 2560, 2560)], acc.at[pl.ds(s * 2560, 2560)])
    pltpu.sync_copy(dst_hbm.at[pl.ds(base, 5120)], dbuf)
    pltpu.sync_copy(typ_hbm.at[pl.ds(base, 5120)], tbuf)
    for k in range(8):
        ones[pl.ds(k * 16, 16)] = jnp.zeros((16,), jnp.float32) + 1.0
    plsc.subcore_barrier()

    def chunk(ch, carry):
        for k in range(8):
            sl = pl.ds(ch * 128 + k * 16, 16)
            ibuf[pl.ds(k * 16, 16)] = dbuf[sl] * 4 + tbuf[sl]
        pltpu.sync_copy(ones, acc.at[plsc.Indices(ibuf.at[:])], add=True)
        return carry

    lax.fori_loop(0, 40, chunk, 0)
    plsc.subcore_barrier()
    pltpu.sync_copy(acc.at[pl.ds(s * 2560, 2560)],
                    cnt_hbm.at[pl.ds(c * 4 * NP + s * 2560, 2560)])


ACC_ROWS = 5120


@functools.partial(
    pl.kernel, mesh=_mesh,
    compiler_params=pltpu.CompilerParams(needs_layout_passes=False),
    out_type=jax.ShapeDtypeStruct((N, EMB), jnp.float32),
    scratch_types=[
        pltpu.VMEM((CE,), jnp.int32),
        pltpu.VMEM((CE,), jnp.int32),
        pltpu.VMEM((CE,), jnp.int32),
        pltpu.VMEM((CE,), jnp.int32),
        pltpu.VMEM((CE,), jnp.int32),
        pltpu.VMEM((CE,), jnp.int32),
        pltpu.VMEM((CE,), jnp.float32),
        pltpu.VMEM((CE, 128), jnp.float32),
        pltpu.VMEM((CE, 128), jnp.float32),
        pltpu.VMEM_SHARED((ACC_ROWS, 128), jnp.float32),
        pltpu.VMEM_SHARED((ACC_ROWS, 128), jnp.float32),
        pltpu.SemaphoreType.DMA,
        pltpu.SemaphoreType.DMA,
        pltpu.SemaphoreType.DMA,
    ])
def _scatter_kernel(src_hbm, dst_hbm, typ_hbm, hpa_hbm, hpb_hbm, dinv_hbm,
                    z_hbm, out_hbm,
                    sbuf, dbuf, tbuf, gbuf, lbuf, nibuf, nbuf, rowsa, rowsb,
                    acca, accb, sema, semb, sem2):
    c = lax.axis_index("c")
    s = lax.axis_index("s")
    half0 = c * HALF
    base = s * (EP // 16)
    pltpu.sync_copy(z_hbm, acca.at[pl.ds(s * 320, 320)])
    pltpu.sync_copy(z_hbm, accb.at[pl.ds(s * 320, 320)])
    plsc.subcore_barrier()

    def chunk(ch, carry):
        off = base + ch * CE
        pltpu.sync_copy(src_hbm.at[pl.ds(off, CE)], sbuf)
        pltpu.sync_copy(dst_hbm.at[pl.ds(off, CE)], dbuf)
        pltpu.sync_copy(typ_hbm.at[pl.ds(off, CE)], tbuf)
        for k in range(CE // 16):
            ol = pl.ds(k * 16, 16)
            sv = sbuf[ol]
            dv = dbuf[ol]
            tv = tbuf[ol]
            valid = tv > 0
            gbuf[ol] = jnp.where(valid, sv * 3 + tv - 1, 0)
            lv = dv - half0
            inh = (lv >= 0) & (lv < HALF)
            lbuf[ol] = jnp.where(inh, lv, HALF)
            nibuf[ol] = jnp.where(valid, dv * 4 + tv, 4 * N)
        cpa = pltpu.async_copy(hpa_hbm.at[plsc.Indices(gbuf.at[:])], rowsa, sema)
        cpb = pltpu.async_copy(hpb_hbm.at[plsc.Indices(gbuf.at[:])], rowsb, semb)
        pltpu.async_copy(dinv_hbm.at[plsc.Indices(nibuf.at[:])], nbuf, sem2).wait()
        cpa.wait()
        cpb.wait()

        def scale(j, carry2):
            ndv = plsc.load_gather(nbuf, [jnp.zeros((16,), jnp.int32) + j])
            for f in range(8):
                fs = pl.ds(f * 16, 16)
                rowsa[j, fs] = rowsa[j, fs] * ndv
                rowsb[j, fs] = rowsb[j, fs] * ndv
            return carry2

        lax.fori_loop(0, CE, scale, 0)
        pltpu.sync_copy(rowsa, acca.at[plsc.Indices(lbuf.at[:])], add=True)
        pltpu.sync_copy(rowsb, accb.at[plsc.Indices(lbuf.at[:])], add=True)
        return carry

    lax.fori_loop(0, EP // 16 // CE, chunk, 0)
    plsc.subcore_barrier()
    start = s * 312
    pltpu.sync_copy(acca.at[pl.ds(start, 312)],
                    out_hbm.at[pl.ds(half0 + start, 312), pl.ds(0, 128)])
    pltpu.sync_copy(accb.at[pl.ds(start, 312)],
                    out_hbm.at[pl.ds(half0 + start, 312), pl.ds(128, 128)])

    @pl.when(s == 15)
    def _tail():
        pltpu.sync_copy(acca.at[pl.ds(4992, 8)],
                        out_hbm.at[pl.ds(half0 + 4992, 8), pl.ds(0, 128)])
        pltpu.sync_copy(accb.at[pl.ds(4992, 8)],
                        out_hbm.at[pl.ds(half0 + 4992, 8), pl.ds(128, 128)])


_BM = 1000
_G = N // _BM


def _k1_body(x_ref, w_ref, cnt_ref, hpa_ref, hpb_ref, self_ref, dinv_ref):
    xb = x_ref[...]
    cnt = cnt_ref[0] + cnt_ref[1]
    dinv = lax.rsqrt(cnt + 2.0)
    acc_self = jnp.zeros((_BM, EMB), jnp.float32)
    for r in range(3):
        h = jnp.dot(xb, w_ref[r], preferred_element_type=jnp.float32)
        hp = h * dinv[:, r + 1][:, None]
        hpa_ref[:, r, :] = hp[:, :128]
        hpb_ref[:, r, :] = hp[:, 128:]
        acc_self = acc_self + 2.0 * dinv[:, r + 1][:, None] * hp
    self_ref[...] = acc_self
    dinv_ref[...] = dinv


def _k1(x, W, cnt):
    return pl.pallas_call(
        _k1_body,
        grid=(_G,),
        in_specs=[
            pl.BlockSpec((_BM, EMB), lambda i: (i, 0)),
            pl.BlockSpec((3, EMB, EMB), lambda i: (0, 0, 0)),
            pl.BlockSpec((2, _BM, 4), lambda i: (0, i, 0)),
        ],
        out_specs=[
            pl.BlockSpec((_BM, 3, 128), lambda i: (i, 0, 0)),
            pl.BlockSpec((_BM, 3, 128), lambda i: (i, 0, 0)),
            pl.BlockSpec((_BM, EMB), lambda i: (i, 0)),
            pl.BlockSpec((_BM, 4), lambda i: (i, 0)),
        ],
        out_shape=[
            jax.ShapeDtypeStruct((N, 3, 128), jnp.float32),
            jax.ShapeDtypeStruct((N, 3, 128), jnp.float32),
            jax.ShapeDtypeStruct((N, EMB), jnp.float32),
            jax.ShapeDtypeStruct((N, 4), jnp.float32),
        ],
    )(x, W, cnt)


def _k2_body(s_ref, self_ref, b_ref, cnt_ref, w_ref, hpa_ref, hpb_ref, self2_ref):
    bsum = jnp.sum(b_ref[...], axis=0, keepdims=True)
    x2 = jnp.maximum(s_ref[...] + self_ref[...] + bsum, 0.0)
    cnt = cnt_ref[0] + cnt_ref[1]
    dinv = lax.rsqrt(cnt + 2.0)
    acc_self = jnp.zeros((_BM, EMB), jnp.float32)
    for r in range(3):
        h = jnp.dot(x2, w_ref[r], preferred_element_type=jnp.float32)
        hp = h * dinv[:, r + 1][:, None]
        hpa_ref[:, r, :] = hp[:, :128]
        hpb_ref[:, r, :] = hp[:, 128:]
        acc_self = acc_self + 2.0 * dinv[:, r + 1][:, None] * hp
    self2_ref[...] = acc_self


def _k2(s1, self1, b1, cnt, W):
    return pl.pallas_call(
        _k2_body,
        grid=(_G,),
        in_specs=[
            pl.BlockSpec((_BM, EMB), lambda i: (i, 0)),
            pl.BlockSpec((_BM, EMB), lambda i: (i, 0)),
            pl.BlockSpec((3, EMB), lambda i: (0, 0)),
            pl.BlockSpec((2, _BM, 4), lambda i: (0, i, 0)),
            pl.BlockSpec((3, EMB, EMB), lambda i: (0, 0, 0)),
        ],
        out_specs=[
            pl.BlockSpec((_BM, 3, 128), lambda i: (i, 0, 0)),
            pl.BlockSpec((_BM, 3, 128), lambda i: (i, 0, 0)),
            pl.BlockSpec((_BM, EMB), lambda i: (i, 0)),
        ],
        out_shape=[
            jax.ShapeDtypeStruct((N, 3, 128), jnp.float32),
            jax.ShapeDtypeStruct((N, 3, 128), jnp.float32),
            jax.ShapeDtypeStruct((N, EMB), jnp.float32),
        ],
    )(s1, self1, b1, cnt, W)


def _k3_body(s_ref, self_ref, b_ref, wc1_ref, bc1_ref, wc2_ref, bc2_ref,
             out2_ref, out_ref):
    bsum = jnp.sum(b_ref[...], axis=0, keepdims=True)
    out2 = s_ref[...] + self_ref[...] + bsum
    z = jnp.maximum(
        jnp.dot(out2, wc1_ref[...], preferred_element_type=jnp.float32)
        + bc1_ref[...], 0.0)
    out_ref[...] = (jnp.dot(z, wc2_ref[...], preferred_element_type=jnp.float32)
                    + bc2_ref[...])
    out2_ref[...] = out2


def _k3(s2, self2, b2, Wc1, bc1, Wc2p, bc2p):
    return pl.pallas_call(
        _k3_body,
        grid=(_G,),
        in_specs=[
            pl.BlockSpec((_BM, EMB), lambda i: (i, 0)),
            pl.BlockSpec((_BM, EMB), lambda i: (i, 0)),
            pl.BlockSpec((3, EMB), lambda i: (0, 0)),
            pl.BlockSpec((EMB, FC), lambda i: (0, 0)),
            pl.BlockSpec((1, FC), lambda i: (0, 0)),
            pl.BlockSpec((FC, 128), lambda i: (0, 0)),
            pl.BlockSpec((1, 128), lambda i: (0, 0)),
        ],
        out_specs=[
            pl.BlockSpec((_BM, EMB), lambda i: (i, 0)),
            pl.BlockSpec((_BM, 128), lambda i: (i, 0)),
        ],
        out_shape=[
            jax.ShapeDtypeStruct((N, EMB), jnp.float32),
            jax.ShapeDtypeStruct((N, 128), jnp.float32),
        ],
    )(s2, self2, b2, Wc1, bc1, Wc2p, bc2p)


def kernel(x, edge_index, edge_type, W1, b1, W2, b2, Wc1, bc1, Wc2, bc2):
    src = edge_index[0]
    dst = edge_index[1]
    pad = EP - E
    zi = jnp.zeros((pad,), jnp.int32)
    srcp = jnp.concatenate([src, zi])
    dstp = jnp.concatenate([dst, zi])
    typp = jnp.concatenate([edge_type, zi])
    z_big = jnp.zeros((320, 128), jnp.float32)
    z_cnt = jnp.zeros((4 * NP,), jnp.float32)

    cntflat = _deg_kernel(dstp, typp, z_cnt)
    cnt = cntflat.reshape(2, NP, 4)

    hp1a, hp1b, self1, dinv4 = _k1(x, W1, cnt)
    dinvflat = jnp.concatenate([dinv4.reshape(-1),
                                jnp.zeros((4,), jnp.float32)])
    s1 = _scatter_kernel(srcp, dstp, typp, hp1a.reshape(3 * N, 128),
                         hp1b.reshape(3 * N, 128), dinvflat, z_big)
    hp2a, hp2b, self2 = _k2(s1, self1, b1, cnt, W2)
    s2 = _scatter_kernel(srcp, dstp, typp, hp2a.reshape(3 * N, 128),
                         hp2b.reshape(3 * N, 128), dinvflat, z_big)

    wc2p = jnp.concatenate([Wc2, jnp.zeros((FC, 128 - 4), jnp.float32)], axis=1)
    bc2p = jnp.concatenate([bc2, jnp.zeros((128 - 4,), jnp.float32)])
    out2, outp = _k3(s2, self2, b2, Wc1, bc1.reshape(1, FC),
                     wc2p, bc2p.reshape(1, 128))
    out = outp[:, :4]

    node_mask = (jax.random.uniform(jax.random.key(1), (N, 1)) > 0.1
                 ).astype(jnp.float32)
    return (out, node_mask, out2)
